# Initial kernel scaffold; baseline (speedup 1.0000x reference)
import jax
import jax.numpy as jnp
from jax.experimental import pallas as pl


def kernel(boxes, scores):
    raise NotImplementedError("write your pallas kernel here")


# speedup vs baseline: 70.5438x; 70.5438x over previous
import functools

import jax
import jax.numpy as jnp
from jax.experimental import pallas as pl
from jax.experimental.pallas import tpu as pltpu

IOU_T = 0.4
CONF_T = 0.2
C = 128


def _nms_body(pk_ref, keep_ref, sup_ref):
    nc = pk_ref.shape[1]
    rows = jax.lax.broadcasted_iota(jnp.int32, (C, C), 0)
    cols = jax.lax.broadcasted_iota(jnp.int32, (C, C), 1)

    sup_ref[...] = jnp.zeros_like(sup_ref)

    def iou_gt(x1r, y1r, x2r, y2r, ar, x1c, y1c, x2c, y2c, ac):
        xx1 = jnp.maximum(x1r, x1c)
        yy1 = jnp.maximum(y1r, y1c)
        xx2 = jnp.minimum(x2r, x2c)
        yy2 = jnp.minimum(y2r, y2c)
        inter = jnp.clip(xx2 - xx1, 0.0) * jnp.clip(yy2 - yy1, 0.0)
        iou = inter / (ar + ac - inter + 1e-9)
        return iou > IOU_T

    def chunk_body(k, _):
        blk = pk_ref[0, k]
        blkt = jnp.transpose(blk)
        x1 = blk[0:1]
        y1 = blk[1:2]
        x2 = blk[2:3]
        y2 = blk[3:4]
        s = blk[4:5]
        ar = blk[5:6]
        x1t = blkt[:, 0:1]
        y1t = blkt[:, 1:2]
        x2t = blkt[:, 2:3]
        y2t = blkt[:, 3:4]
        st = blkt[:, 4:5]
        art = blkt[:, 5:6]

        g = iou_gt(x1t, y1t, x2t, y2t, art, x1, y1, x2, y2, ar)
        gt = jnp.transpose(g)
        s_rl = (g & (rows < cols)).astype(jnp.float32)
        s_lr = (gt & (cols < rows)).astype(jnp.float32)

        pre = sup_ref[k] > 0
        valid_l = (s > CONF_T) & ~pre
        valid_r = (st > CONF_T) & ~jnp.transpose(pre)

        valid_lf = valid_l.astype(jnp.float32)
        valid_rf = valid_r.astype(jnp.float32)

        def fix_cond(st_):
            return st_[0]

        def fix_body(st_):
            _, keep_lf = st_
            sup_r = jnp.max(s_lr * keep_lf, axis=1, keepdims=True)
            keep_rf_ = valid_rf * (1.0 - jnp.minimum(sup_r, 1.0))
            sup_l = jnp.max(s_rl * keep_rf_, axis=0, keepdims=True)
            keep_lf2 = valid_lf * (1.0 - jnp.minimum(sup_l, 1.0))
            changed = jnp.any(keep_lf2 != keep_lf)
            return changed, keep_lf2

        _, keep_lf = jax.lax.while_loop(
            fix_cond, fix_body, (jnp.bool_(True), valid_lf))
        keep_l = keep_lf > 0.0
        keep_rf = valid_rf * (1.0 - jnp.minimum(
            jnp.max(s_lr * keep_lf, axis=1, keepdims=True), 1.0))

        tiny = ((x2 - x1) >= 1.0) & ((y2 - y1) >= 1.0)
        keep_ref[0, k] = (keep_l & tiny).astype(jnp.float32)

        def cross_body(m, _):
            b2 = pk_ref[0, m]
            cs = iou_gt(x1t, y1t, x2t, y2t, art,
                        b2[0:1], b2[1:2], b2[2:3], b2[3:4], b2[5:6])
            supm = jnp.max(cs.astype(jnp.float32) * keep_rf,
                           axis=0, keepdims=True)
            sup_ref[m] = jnp.maximum(sup_ref[m], supm)
            return 0

        jax.lax.fori_loop(k + 1, nc, cross_body, 0)
        return 0

    jax.lax.fori_loop(0, nc, chunk_body, 0)


@jax.jit
def kernel(boxes, scores):
    B, N = scores.shape
    nc = (N + C - 1) // C
    npad = nc * C

    order = jnp.argsort(-scores, axis=1)
    bs = jnp.take_along_axis(boxes, order[..., None], axis=1)
    ss = jnp.take_along_axis(scores, order, axis=1)
    areas = (bs[:, :, 2] - bs[:, :, 0]) * (bs[:, :, 3] - bs[:, :, 1])

    pad = npad - N
    x1 = jnp.pad(bs[:, :, 0], ((0, 0), (0, pad)))
    y1 = jnp.pad(bs[:, :, 1], ((0, 0), (0, pad)))
    x2 = jnp.pad(bs[:, :, 2], ((0, 0), (0, pad)))
    y2 = jnp.pad(bs[:, :, 3], ((0, 0), (0, pad)))
    sp = jnp.pad(ss, ((0, 0), (0, pad)), constant_values=-1.0)
    ap = jnp.pad(areas, ((0, 0), (0, pad)))
    z = jnp.zeros_like(x1)
    pk = jnp.stack([x1, y1, x2, y2, sp, ap, z, z], axis=1)
    pk = pk.reshape(B, 8, nc, C).transpose(0, 2, 1, 3)

    keep_sorted = pl.pallas_call(
        _nms_body,
        grid=(B,),
        in_specs=[pl.BlockSpec((1, nc, 8, C), lambda b: (b, 0, 0, 0))],
        out_specs=pl.BlockSpec((1, nc, 1, C), lambda b: (b, 0, 0, 0)),
        out_shape=jax.ShapeDtypeStruct((B, nc, 1, C), jnp.float32),
        scratch_shapes=[pltpu.VMEM((nc, 1, C), jnp.float32)],
    )(pk)

    keep_sorted = keep_sorted.reshape(B, npad)[:, :N]
    bidx = jnp.arange(B)[:, None]
    keep = jnp.zeros((B, N), boxes.dtype).at[bidx, order].set(keep_sorted)
    out = jnp.concatenate(
        [boxes * keep[..., None], (scores * keep)[..., None]], axis=-1)
    return out
